# baseline (speedup 1.0000x reference)
import functools

import jax
import jax.numpy as jnp
from jax import lax
from jax.experimental import pallas as pl
from jax.experimental.pallas import tpu as pltpu
from jax.experimental.pallas import tpu_sc as plsc

_NC = 2
_NS = 16
_NW = _NC * _NS
_LANES = 16

_SELU_ALPHA = 1.6732632423543772
_SELU_SCALE = 1.0507009873554805


def _sc_mesh():
    return plsc.VectorSubcoreMesh(core_axis_name="c", subcore_axis_name="s")


def _deg_partials(col, ew, n_pad):
    e = col.shape[0]
    epw = e // _NW

    @functools.partial(
        pl.kernel,
        out_type=jax.ShapeDtypeStruct((_NW, n_pad), jnp.float32),
        mesh=_sc_mesh(),
        compiler_params=pltpu.CompilerParams(needs_layout_passes=False),
        scratch_types=[
            pltpu.VMEM((epw,), jnp.int32),
            pltpu.VMEM((epw,), jnp.float32),
            pltpu.VMEM((n_pad,), jnp.float32),
        ],
    )
    def deg_kernel(col_hbm, ew_hbm, out_hbm, col_v, ew_v, acc_v):
        wid = lax.axis_index("s") * _NC + lax.axis_index("c")
        base = wid * epw
        pltpu.sync_copy(col_hbm.at[pl.ds(base, epw)], col_v)
        pltpu.sync_copy(ew_hbm.at[pl.ds(base, epw)], ew_v)

        zeros = jnp.zeros((_LANES,), jnp.float32)

        def zbody(i, carry):
            acc_v[pl.ds(i * _LANES, _LANES)] = zeros
            return carry

        lax.fori_loop(0, n_pad // _LANES, zbody, 0)

        def body(i, carry):
            idx = col_v[pl.ds(i * _LANES, _LANES)]
            w = ew_v[pl.ds(i * _LANES, _LANES)]
            plsc.addupdate_scatter(acc_v, [idx], w)
            return carry

        lax.fori_loop(0, epw // _LANES, body, 0)
        pltpu.sync_copy(acc_v, out_hbm.at[wid])

    return deg_kernel(col, ew)


def _edge_aggregate(g_t, row, col, ew, n_pad, feats):
    e = row.shape[0]
    fpw = feats // _NW
    eb = 6400
    nblk = e // eb
    npair = nblk // 2
    unroll = 16

    @functools.partial(
        pl.kernel,
        out_type=jax.ShapeDtypeStruct((feats, n_pad), jnp.float32),
        mesh=_sc_mesh(),
        compiler_params=pltpu.CompilerParams(needs_layout_passes=False),
        scratch_types=(
            [pltpu.VMEM((n_pad,), jnp.float32) for _ in range(2 * fpw)]
            + [
                pltpu.VMEM((eb,), jnp.int32),
                pltpu.VMEM((eb,), jnp.int32),
                pltpu.VMEM((eb,), jnp.float32),
                pltpu.VMEM((eb,), jnp.int32),
                pltpu.VMEM((eb,), jnp.int32),
                pltpu.VMEM((eb,), jnp.float32),
                pltpu.SemaphoreType.DMA,
                pltpu.SemaphoreType.DMA,
            ]
        ),
    )
    def msg_kernel(g_hbm, row_hbm, col_hbm, ew_hbm, out_hbm, *scratch):
        g_vs = scratch[:fpw]
        acc_vs = scratch[fpw:2 * fpw]
        b0 = scratch[2 * fpw:2 * fpw + 3] + (scratch[2 * fpw + 6],)
        b1 = scratch[2 * fpw + 3:2 * fpw + 6] + (scratch[2 * fpw + 7],)
        bufs = (b0, b1)
        wid = lax.axis_index("s") * _NC + lax.axis_index("c")
        fbase = wid * fpw
        for f in range(fpw):
            pltpu.sync_copy(g_hbm.at[fbase + f], g_vs[f])

        zeros = jnp.zeros((_LANES,), jnp.float32)

        def zbody(i, carry):
            for f in range(fpw):
                acc_vs[f][pl.ds(i * _LANES, _LANES)] = zeros
            return carry

        lax.fori_loop(0, n_pad // _LANES, zbody, 0)

        def start(slot, b):
            rv, cv, wv, sem = bufs[slot]
            ebase = b * eb
            pltpu.async_copy(row_hbm.at[pl.ds(ebase, eb)], rv, sem)
            pltpu.async_copy(col_hbm.at[pl.ds(ebase, eb)], cv, sem)
            pltpu.async_copy(ew_hbm.at[pl.ds(ebase, eb)], wv, sem)

        def wait(slot):
            rv, cv, wv, sem = bufs[slot]
            pltpu.make_async_copy(row_hbm.at[pl.ds(0, eb)], rv, sem).wait()
            pltpu.make_async_copy(col_hbm.at[pl.ds(0, eb)], cv, sem).wait()
            pltpu.make_async_copy(ew_hbm.at[pl.ds(0, eb)], wv, sem).wait()

        def process(slot):
            rv, cv, wv, _ = bufs[slot]

            @plsc.parallel_loop(0, eb // _LANES, 1, unroll=unroll)
            def _chunks(i):
                off = i * _LANES
                r = rv[pl.ds(off, _LANES)]
                c = cv[pl.ds(off, _LANES)]
                w = wv[pl.ds(off, _LANES)]
                for f in range(fpw):
                    v = plsc.load_gather(g_vs[f], [r])
                    plsc.addupdate_scatter(acc_vs[f], [c], v * w)

        start(0, 0)

        def pair(i, carry):
            start(1, 2 * i + 1)
            wait(0)
            process(0)

            @pl.when(i < npair - 1)
            def _():
                start(0, 2 * i + 2)

            wait(1)
            process(1)
            return carry

        lax.fori_loop(0, npair, pair, 0)
        for f in range(fpw):
            pltpu.sync_copy(acc_vs[f], out_hbm.at[fbase + f])

    return msg_kernel(g_t, row, col, ew)


def _embed_matmul(x_t, w0_t, b0c, w1_t, n_pad):
    bn = 1280
    grid = n_pad // bn
    emb = w0_t.shape[0]
    feats = w1_t.shape[0]
    in_ch = x_t.shape[0]

    def body(x_ref, w0_ref, b0_ref, w1_ref, out_ref):
        h = jnp.dot(w0_ref[...], x_ref[...], preferred_element_type=jnp.float32)
        h = h + b0_ref[...]
        h = _SELU_SCALE * jnp.where(h > 0, h, _SELU_ALPHA * (jnp.exp(h) - 1.0))
        out_ref[...] = jnp.dot(w1_ref[...], h, preferred_element_type=jnp.float32)

    return pl.pallas_call(
        body,
        grid=(grid,),
        in_specs=[
            pl.BlockSpec((in_ch, bn), lambda i: (0, i)),
            pl.BlockSpec((emb, in_ch), lambda i: (0, 0)),
            pl.BlockSpec((emb, 1), lambda i: (0, 0)),
            pl.BlockSpec((feats, emb), lambda i: (0, 0)),
        ],
        out_specs=pl.BlockSpec((feats, bn), lambda i: (0, i)),
        out_shape=jax.ShapeDtypeStruct((feats, n_pad), jnp.float32),
    )(x_t, w0_t, b0c, w1_t)


def _dis_and_g1(parts, hw1_t):
    nw, n_pad = parts.shape
    feats = hw1_t.shape[0]

    def body(p_ref, h_ref, dis_ref, g_ref):
        deg = jnp.sum(p_ref[...], axis=0, keepdims=True) + 1.0
        dis = lax.rsqrt(deg)
        dis_ref[...] = dis
        g_ref[...] = h_ref[...] * dis

    return pl.pallas_call(
        body,
        out_shape=(
            jax.ShapeDtypeStruct((1, n_pad), jnp.float32),
            jax.ShapeDtypeStruct((feats, n_pad), jnp.float32),
        ),
    )(parts, hw1_t)


def _conv1_epilogue(s1_t, g1_t, dis, w2_t, b1c):
    feats, n_pad = g1_t.shape

    def body(s_ref, g_ref, d_ref, w2_ref, b1_ref, out_ref):
        d = d_ref[...]
        out1 = d * (s_ref[...] + g_ref[...]) + b1_ref[...]
        hw2 = jnp.dot(w2_ref[...], out1, preferred_element_type=jnp.float32)
        out_ref[...] = hw2 * d

    return pl.pallas_call(
        body,
        out_shape=jax.ShapeDtypeStruct((feats, n_pad), jnp.float32),
    )(s1_t, g1_t, dis, w2_t, b1c)


def _conv2_softmax(s2_t, g2_t, dis, b2c):
    feats, n_pad = g2_t.shape

    def body(s_ref, g_ref, d_ref, b2_ref, out_ref):
        o = d_ref[...] * (s_ref[...] + g_ref[...]) + b2_ref[...]
        m = jnp.max(o, axis=0, keepdims=True)
        ex = jnp.exp(o - m)
        out_ref[...] = ex / jnp.sum(ex, axis=0, keepdims=True)

    return pl.pallas_call(
        body,
        out_shape=jax.ShapeDtypeStruct((feats, n_pad), jnp.float32),
    )(s2_t, g2_t, dis, b2c)


def kernel(x, edge_index, edge_attr, W0, b0, W1, b1, W2, b2):
    n = x.shape[0]
    n_pad = 10240
    feats = W1.shape[1]

    row = edge_index[0]
    col = edge_index[1]
    ew = edge_attr

    x_t = jnp.pad(x.T, ((0, 0), (0, n_pad - n)))
    w0_t = W0.T
    w1_t = W1.T
    w2_t = W2.T
    b0c = b0[:, None]
    b1c = b1[:, None]
    b2c = b2[:, None]

    parts = _deg_partials(col, ew, n_pad)
    hw1_t = _embed_matmul(x_t, w0_t, b0c, w1_t, n_pad)

    dis, g1_t = _dis_and_g1(parts, hw1_t)
    s1_t = _edge_aggregate(g1_t, row, col, ew, n_pad, feats)
    g2_t = _conv1_epilogue(s1_t, g1_t, dis, w2_t, b1c)
    s2_t = _edge_aggregate(g2_t, row, col, ew, n_pad, feats)
    out_t = _conv2_softmax(s2_t, g2_t, dis, b2c)

    return out_t[:, :n].T


# speedup vs baseline: 1.1348x; 1.1348x over previous
import functools

import jax
import jax.numpy as jnp
from jax import lax
from jax.experimental import pallas as pl
from jax.experimental.pallas import tpu as pltpu
from jax.experimental.pallas import tpu_sc as plsc

_NC = 2
_NS = 16
_NW = _NC * _NS
_LANES = 16

_SELU_ALPHA = 1.6732632423543772
_SELU_SCALE = 1.0507009873554805


def _sc_mesh():
    return plsc.VectorSubcoreMesh(core_axis_name="c", subcore_axis_name="s")


def _deg_partials(col, ew, n_pad):
    e = col.shape[0]
    epw = e // _NW

    @functools.partial(
        pl.kernel,
        out_type=jax.ShapeDtypeStruct((_NW, n_pad), jnp.float32),
        mesh=_sc_mesh(),
        compiler_params=pltpu.CompilerParams(needs_layout_passes=False),
        scratch_types=[
            pltpu.VMEM((epw,), jnp.int32),
            pltpu.VMEM((epw,), jnp.float32),
            pltpu.VMEM((n_pad,), jnp.float32),
        ],
    )
    def deg_kernel(col_hbm, ew_hbm, out_hbm, col_v, ew_v, acc_v):
        wid = lax.axis_index("s") * _NC + lax.axis_index("c")
        base = wid * epw
        pltpu.sync_copy(col_hbm.at[pl.ds(base, epw)], col_v)
        pltpu.sync_copy(ew_hbm.at[pl.ds(base, epw)], ew_v)

        zeros = jnp.zeros((_LANES,), jnp.float32)

        def zbody(i, carry):
            acc_v[pl.ds(i * _LANES, _LANES)] = zeros
            return carry

        lax.fori_loop(0, n_pad // _LANES, zbody, 0)

        def body(i, carry):
            idx = col_v[pl.ds(i * _LANES, _LANES)]
            w = ew_v[pl.ds(i * _LANES, _LANES)]
            plsc.addupdate_scatter(acc_v, [idx], w)
            return carry

        lax.fori_loop(0, epw // _LANES, body, 0)
        pltpu.sync_copy(acc_v, out_hbm.at[wid])

    return deg_kernel(col, ew)


def _edge_aggregate(g_t, row, col, ew, n_pad, feats):
    e = row.shape[0]
    nhalf = 2
    fpw = feats * nhalf // _NW
    ehalf_sz = e // nhalf
    eb = 3200
    nblk = ehalf_sz // eb
    npair = nblk // 2
    unroll = 8

    @functools.partial(
        pl.kernel,
        out_type=jax.ShapeDtypeStruct((nhalf, feats, n_pad), jnp.float32),
        mesh=_sc_mesh(),
        compiler_params=pltpu.CompilerParams(needs_layout_passes=False),
        scratch_types=(
            [pltpu.VMEM((n_pad,), jnp.float32) for _ in range(2 * fpw)]
            + [
                pltpu.VMEM((eb,), jnp.int32),
                pltpu.VMEM((eb,), jnp.int32),
                pltpu.VMEM((eb,), jnp.float32),
                pltpu.VMEM((eb,), jnp.int32),
                pltpu.VMEM((eb,), jnp.int32),
                pltpu.VMEM((eb,), jnp.float32),
                pltpu.SemaphoreType.DMA,
                pltpu.SemaphoreType.DMA,
            ]
        ),
    )
    def msg_kernel(g_hbm, row_hbm, col_hbm, ew_hbm, out_hbm, *scratch):
        g_vs = scratch[:fpw]
        acc_vs = scratch[fpw:2 * fpw]
        b0 = scratch[2 * fpw:2 * fpw + 3] + (scratch[2 * fpw + 6],)
        b1 = scratch[2 * fpw + 3:2 * fpw + 6] + (scratch[2 * fpw + 7],)
        bufs = (b0, b1)
        wid = lax.axis_index("s") * _NC + lax.axis_index("c")
        ehalf = wid % nhalf
        fbase = (wid // nhalf) * fpw
        ebase0 = ehalf * ehalf_sz
        for f in range(fpw):
            pltpu.sync_copy(g_hbm.at[fbase + f], g_vs[f])

        zeros = jnp.zeros((_LANES,), jnp.float32)

        def zbody(i, carry):
            for f in range(fpw):
                acc_vs[f][pl.ds(i * _LANES, _LANES)] = zeros
            return carry

        lax.fori_loop(0, n_pad // _LANES, zbody, 0)

        def start(slot, b):
            rv, cv, wv, sem = bufs[slot]
            ebase = ebase0 + b * eb
            pltpu.async_copy(row_hbm.at[pl.ds(ebase, eb)], rv, sem)
            pltpu.async_copy(col_hbm.at[pl.ds(ebase, eb)], cv, sem)
            pltpu.async_copy(ew_hbm.at[pl.ds(ebase, eb)], wv, sem)

        def wait(slot):
            rv, cv, wv, sem = bufs[slot]
            pltpu.make_async_copy(row_hbm.at[pl.ds(0, eb)], rv, sem).wait()
            pltpu.make_async_copy(col_hbm.at[pl.ds(0, eb)], cv, sem).wait()
            pltpu.make_async_copy(ew_hbm.at[pl.ds(0, eb)], wv, sem).wait()

        def process(slot):
            rv, cv, wv, _ = bufs[slot]

            @plsc.parallel_loop(0, eb // _LANES, 1, unroll=unroll)
            def _chunks(i):
                off = i * _LANES
                r = rv[pl.ds(off, _LANES)]
                c = cv[pl.ds(off, _LANES)]
                w = wv[pl.ds(off, _LANES)]
                for f in range(fpw):
                    v = plsc.load_gather(g_vs[f], [r])
                    plsc.addupdate_scatter(acc_vs[f], [c], v * w)

        start(0, 0)

        def pair(i, carry):
            start(1, 2 * i + 1)
            wait(0)
            process(0)

            @pl.when(i < npair - 1)
            def _():
                start(0, 2 * i + 2)

            wait(1)
            process(1)
            return carry

        lax.fori_loop(0, npair, pair, 0)
        for f in range(fpw):
            pltpu.sync_copy(acc_vs[f], out_hbm.at[ehalf, fbase + f])

    return msg_kernel(g_t, row, col, ew)


def _embed_matmul(x_t, w0_t, b0c, w1_t, n_pad):
    bn = 1280
    grid = n_pad // bn
    emb = w0_t.shape[0]
    feats = w1_t.shape[0]
    in_ch = x_t.shape[0]

    def body(x_ref, w0_ref, b0_ref, w1_ref, out_ref):
        h = jnp.dot(w0_ref[...], x_ref[...], preferred_element_type=jnp.float32)
        h = h + b0_ref[...]
        h = _SELU_SCALE * jnp.where(h > 0, h, _SELU_ALPHA * (jnp.exp(h) - 1.0))
        out_ref[...] = jnp.dot(w1_ref[...], h, preferred_element_type=jnp.float32)

    return pl.pallas_call(
        body,
        grid=(grid,),
        in_specs=[
            pl.BlockSpec((in_ch, bn), lambda i: (0, i)),
            pl.BlockSpec((emb, in_ch), lambda i: (0, 0)),
            pl.BlockSpec((emb, 1), lambda i: (0, 0)),
            pl.BlockSpec((feats, emb), lambda i: (0, 0)),
        ],
        out_specs=pl.BlockSpec((feats, bn), lambda i: (0, i)),
        out_shape=jax.ShapeDtypeStruct((feats, n_pad), jnp.float32),
    )(x_t, w0_t, b0c, w1_t)


def _dis_and_g1(parts, hw1_t):
    nw, n_pad = parts.shape
    feats = hw1_t.shape[0]

    def body(p_ref, h_ref, dis_ref, g_ref):
        deg = jnp.sum(p_ref[...], axis=0, keepdims=True) + 1.0
        dis = lax.rsqrt(deg)
        dis_ref[...] = dis
        g_ref[...] = h_ref[...] * dis

    return pl.pallas_call(
        body,
        out_shape=(
            jax.ShapeDtypeStruct((1, n_pad), jnp.float32),
            jax.ShapeDtypeStruct((feats, n_pad), jnp.float32),
        ),
    )(parts, hw1_t)


def _conv1_epilogue(s1_t, g1_t, dis, w2_t, b1c):
    feats, n_pad = g1_t.shape

    def body(s_ref, g_ref, d_ref, w2_ref, b1_ref, out_ref):
        d = d_ref[...]
        s = s_ref[0] + s_ref[1]
        out1 = d * (s + g_ref[...]) + b1_ref[...]
        hw2 = jnp.dot(w2_ref[...], out1, preferred_element_type=jnp.float32)
        out_ref[...] = hw2 * d

    return pl.pallas_call(
        body,
        out_shape=jax.ShapeDtypeStruct((feats, n_pad), jnp.float32),
    )(s1_t, g1_t, dis, w2_t, b1c)


def _conv2_softmax(s2_t, g2_t, dis, b2c):
    feats, n_pad = g2_t.shape

    def body(s_ref, g_ref, d_ref, b2_ref, out_ref):
        o = d_ref[...] * (s_ref[0] + s_ref[1] + g_ref[...]) + b2_ref[...]
        m = jnp.max(o, axis=0, keepdims=True)
        ex = jnp.exp(o - m)
        out_ref[...] = ex / jnp.sum(ex, axis=0, keepdims=True)

    return pl.pallas_call(
        body,
        out_shape=jax.ShapeDtypeStruct((feats, n_pad), jnp.float32),
    )(s2_t, g2_t, dis, b2c)


def kernel(x, edge_index, edge_attr, W0, b0, W1, b1, W2, b2):
    n = x.shape[0]
    n_pad = 10240
    feats = W1.shape[1]

    row = edge_index[0]
    col = edge_index[1]
    ew = edge_attr

    x_t = jnp.pad(x.T, ((0, 0), (0, n_pad - n)))
    w0_t = W0.T
    w1_t = W1.T
    w2_t = W2.T
    b0c = b0[:, None]
    b1c = b1[:, None]
    b2c = b2[:, None]

    parts = _deg_partials(col, ew, n_pad)
    hw1_t = _embed_matmul(x_t, w0_t, b0c, w1_t, n_pad)

    dis, g1_t = _dis_and_g1(parts, hw1_t)
    s1_t = _edge_aggregate(g1_t, row, col, ew, n_pad, feats)
    g2_t = _conv1_epilogue(s1_t, g1_t, dis, w2_t, b1c)
    s2_t = _edge_aggregate(g2_t, row, col, ew, n_pad, feats)
    out_t = _conv2_softmax(s2_t, g2_t, dis, b2c)

    return out_t[:, :n].T


# speedup vs baseline: 1.1580x; 1.0204x over previous
import functools

import jax
import jax.numpy as jnp
from jax import lax
from jax.experimental import pallas as pl
from jax.experimental.pallas import tpu as pltpu
from jax.experimental.pallas import tpu_sc as plsc

_NC = 2
_NS = 16
_NW = _NC * _NS
_LANES = 16

_SELU_ALPHA = 1.6732632423543772
_SELU_SCALE = 1.0507009873554805


def _sc_mesh():
    return plsc.VectorSubcoreMesh(core_axis_name="c", subcore_axis_name="s")


def _deg_partials(col, ew, n_pad):
    e = col.shape[0]
    epw = e // _NW

    @functools.partial(
        pl.kernel,
        out_type=jax.ShapeDtypeStruct((_NW, n_pad), jnp.float32),
        mesh=_sc_mesh(),
        compiler_params=pltpu.CompilerParams(needs_layout_passes=False),
        scratch_types=[
            pltpu.VMEM((epw,), jnp.int32),
            pltpu.VMEM((epw,), jnp.float32),
            pltpu.VMEM((n_pad,), jnp.float32),
        ],
    )
    def deg_kernel(col_hbm, ew_hbm, out_hbm, col_v, ew_v, acc_v):
        wid = lax.axis_index("s") * _NC + lax.axis_index("c")
        base = wid * epw
        pltpu.sync_copy(col_hbm.at[pl.ds(base, epw)], col_v)
        pltpu.sync_copy(ew_hbm.at[pl.ds(base, epw)], ew_v)

        zeros = jnp.zeros((_LANES,), jnp.float32)

        def zbody(i, carry):
            acc_v[pl.ds(i * _LANES, _LANES)] = zeros
            return carry

        lax.fori_loop(0, n_pad // _LANES, zbody, 0)

        def body(i, carry):
            idx = col_v[pl.ds(i * _LANES, _LANES)]
            w = ew_v[pl.ds(i * _LANES, _LANES)]
            plsc.addupdate_scatter(acc_v, [idx], w)
            return carry

        lax.fori_loop(0, epw // _LANES, body, 0)
        pltpu.sync_copy(acc_v, out_hbm.at[wid])

    return deg_kernel(col, ew)


def _edge_aggregate(g_t, row, col, ew, n_pad, feats):
    e = row.shape[0]
    nhalf = 2
    fpw = feats * nhalf // _NW
    ehalf_sz = e // nhalf
    eb = 3200
    nblk = ehalf_sz // eb
    npair = nblk // 2
    unroll = 4

    @functools.partial(
        pl.kernel,
        out_type=jax.ShapeDtypeStruct((nhalf, feats, n_pad), jnp.float32),
        mesh=_sc_mesh(),
        compiler_params=pltpu.CompilerParams(needs_layout_passes=False),
        scratch_types=(
            [pltpu.VMEM((n_pad,), jnp.float32) for _ in range(2 * fpw)]
            + [
                pltpu.VMEM((eb,), jnp.int32),
                pltpu.VMEM((eb,), jnp.int32),
                pltpu.VMEM((eb,), jnp.float32),
                pltpu.VMEM((eb,), jnp.int32),
                pltpu.VMEM((eb,), jnp.int32),
                pltpu.VMEM((eb,), jnp.float32),
                pltpu.SemaphoreType.DMA,
                pltpu.SemaphoreType.DMA,
            ]
        ),
    )
    def msg_kernel(g_hbm, row_hbm, col_hbm, ew_hbm, out_hbm, *scratch):
        g_vs = scratch[:fpw]
        acc_vs = scratch[fpw:2 * fpw]
        b0 = scratch[2 * fpw:2 * fpw + 3] + (scratch[2 * fpw + 6],)
        b1 = scratch[2 * fpw + 3:2 * fpw + 6] + (scratch[2 * fpw + 7],)
        bufs = (b0, b1)
        wid = lax.axis_index("s") * _NC + lax.axis_index("c")
        ehalf = wid % nhalf
        fbase = (wid // nhalf) * fpw
        ebase0 = ehalf * ehalf_sz
        for f in range(fpw):
            pltpu.sync_copy(g_hbm.at[fbase + f], g_vs[f])

        zeros = jnp.zeros((_LANES,), jnp.float32)

        def zbody(i, carry):
            for f in range(fpw):
                acc_vs[f][pl.ds(i * _LANES, _LANES)] = zeros
            return carry

        lax.fori_loop(0, n_pad // _LANES, zbody, 0)

        def start(slot, b):
            rv, cv, wv, sem = bufs[slot]
            ebase = ebase0 + b * eb
            pltpu.async_copy(row_hbm.at[pl.ds(ebase, eb)], rv, sem)
            pltpu.async_copy(col_hbm.at[pl.ds(ebase, eb)], cv, sem)
            pltpu.async_copy(ew_hbm.at[pl.ds(ebase, eb)], wv, sem)

        def wait(slot):
            rv, cv, wv, sem = bufs[slot]
            pltpu.make_async_copy(row_hbm.at[pl.ds(0, eb)], rv, sem).wait()
            pltpu.make_async_copy(col_hbm.at[pl.ds(0, eb)], cv, sem).wait()
            pltpu.make_async_copy(ew_hbm.at[pl.ds(0, eb)], wv, sem).wait()

        def process(slot):
            rv, cv, wv, _ = bufs[slot]

            @plsc.parallel_loop(0, eb // _LANES, 1, unroll=unroll)
            def _chunks(i):
                off = i * _LANES
                r = rv[pl.ds(off, _LANES)]
                c = cv[pl.ds(off, _LANES)]
                w = wv[pl.ds(off, _LANES)]
                for f in range(fpw):
                    v = plsc.load_gather(g_vs[f], [r])
                    plsc.addupdate_scatter(acc_vs[f], [c], v * w)

        start(0, 0)

        def pair(i, carry):
            start(1, 2 * i + 1)
            wait(0)
            process(0)

            @pl.when(i < npair - 1)
            def _():
                start(0, 2 * i + 2)

            wait(1)
            process(1)
            return carry

        lax.fori_loop(0, npair, pair, 0)
        for f in range(fpw):
            pltpu.sync_copy(acc_vs[f], out_hbm.at[ehalf, fbase + f])

    return msg_kernel(g_t, row, col, ew)


def _embed_matmul(x_t, w0_t, b0c, w1_t, n_pad):
    bn = 1280
    grid = n_pad // bn
    emb = w0_t.shape[0]
    feats = w1_t.shape[0]
    in_ch = x_t.shape[0]

    def body(x_ref, w0_ref, b0_ref, w1_ref, out_ref):
        h = jnp.dot(w0_ref[...], x_ref[...], preferred_element_type=jnp.float32)
        h = h + b0_ref[...]
        h = _SELU_SCALE * jnp.where(h > 0, h, _SELU_ALPHA * (jnp.exp(h) - 1.0))
        out_ref[...] = jnp.dot(w1_ref[...], h, preferred_element_type=jnp.float32)

    return pl.pallas_call(
        body,
        grid=(grid,),
        in_specs=[
            pl.BlockSpec((in_ch, bn), lambda i: (0, i)),
            pl.BlockSpec((emb, in_ch), lambda i: (0, 0)),
            pl.BlockSpec((emb, 1), lambda i: (0, 0)),
            pl.BlockSpec((feats, emb), lambda i: (0, 0)),
        ],
        out_specs=pl.BlockSpec((feats, bn), lambda i: (0, i)),
        out_shape=jax.ShapeDtypeStruct((feats, n_pad), jnp.float32),
    )(x_t, w0_t, b0c, w1_t)


def _dis_and_g1(parts, hw1_t):
    nw, n_pad = parts.shape
    feats = hw1_t.shape[0]

    def body(p_ref, h_ref, dis_ref, g_ref):
        deg = jnp.sum(p_ref[...], axis=0, keepdims=True) + 1.0
        dis = lax.rsqrt(deg)
        dis_ref[...] = dis
        g_ref[...] = h_ref[...] * dis

    return pl.pallas_call(
        body,
        out_shape=(
            jax.ShapeDtypeStruct((1, n_pad), jnp.float32),
            jax.ShapeDtypeStruct((feats, n_pad), jnp.float32),
        ),
    )(parts, hw1_t)


def _conv1_epilogue(s1_t, g1_t, dis, w2_t, b1c):
    feats, n_pad = g1_t.shape

    def body(s_ref, g_ref, d_ref, w2_ref, b1_ref, out_ref):
        d = d_ref[...]
        s = s_ref[0] + s_ref[1]
        out1 = d * (s + g_ref[...]) + b1_ref[...]
        hw2 = jnp.dot(w2_ref[...], out1, preferred_element_type=jnp.float32)
        out_ref[...] = hw2 * d

    return pl.pallas_call(
        body,
        out_shape=jax.ShapeDtypeStruct((feats, n_pad), jnp.float32),
    )(s1_t, g1_t, dis, w2_t, b1c)


def _conv2_softmax(s2_t, g2_t, dis, b2c):
    feats, n_pad = g2_t.shape

    def body(s_ref, g_ref, d_ref, b2_ref, out_ref):
        o = d_ref[...] * (s_ref[0] + s_ref[1] + g_ref[...]) + b2_ref[...]
        m = jnp.max(o, axis=0, keepdims=True)
        ex = jnp.exp(o - m)
        out_ref[...] = ex / jnp.sum(ex, axis=0, keepdims=True)

    return pl.pallas_call(
        body,
        out_shape=jax.ShapeDtypeStruct((feats, n_pad), jnp.float32),
    )(s2_t, g2_t, dis, b2c)


def kernel(x, edge_index, edge_attr, W0, b0, W1, b1, W2, b2):
    n = x.shape[0]
    n_pad = 10240
    feats = W1.shape[1]

    row = edge_index[0]
    col = edge_index[1]
    ew = edge_attr

    x_t = jnp.pad(x.T, ((0, 0), (0, n_pad - n)))
    w0_t = W0.T
    w1_t = W1.T
    w2_t = W2.T
    b0c = b0[:, None]
    b1c = b1[:, None]
    b2c = b2[:, None]

    parts = _deg_partials(col, ew, n_pad)
    hw1_t = _embed_matmul(x_t, w0_t, b0c, w1_t, n_pad)

    dis, g1_t = _dis_and_g1(parts, hw1_t)
    s1_t = _edge_aggregate(g1_t, row, col, ew, n_pad, feats)
    g2_t = _conv1_epilogue(s1_t, g1_t, dis, w2_t, b1c)
    s2_t = _edge_aggregate(g2_t, row, col, ew, n_pad, feats)
    out_t = _conv2_softmax(s2_t, g2_t, dis, b2c)

    return out_t[:, :n].T


# speedup vs baseline: 1.1664x; 1.0073x over previous
import functools

import jax
import jax.numpy as jnp
from jax import lax
from jax.experimental import pallas as pl
from jax.experimental.pallas import tpu as pltpu
from jax.experimental.pallas import tpu_sc as plsc

_NC = 2
_NS = 16
_NW = _NC * _NS
_LANES = 16

_SELU_ALPHA = 1.6732632423543772
_SELU_SCALE = 1.0507009873554805


def _sc_mesh():
    return plsc.VectorSubcoreMesh(core_axis_name="c", subcore_axis_name="s")


def _deg_partials(col, ew, n_pad):
    e = col.shape[0]
    epw = e // _NW

    @functools.partial(
        pl.kernel,
        out_type=jax.ShapeDtypeStruct((_NW, n_pad), jnp.float32),
        mesh=_sc_mesh(),
        compiler_params=pltpu.CompilerParams(needs_layout_passes=False),
        scratch_types=[
            pltpu.VMEM((epw,), jnp.int32),
            pltpu.VMEM((epw,), jnp.float32),
            pltpu.VMEM((n_pad,), jnp.float32),
        ],
    )
    def deg_kernel(col_hbm, ew_hbm, out_hbm, col_v, ew_v, acc_v):
        wid = lax.axis_index("s") * _NC + lax.axis_index("c")
        base = wid * epw
        pltpu.sync_copy(col_hbm.at[pl.ds(base, epw)], col_v)
        pltpu.sync_copy(ew_hbm.at[pl.ds(base, epw)], ew_v)

        zeros = jnp.zeros((_LANES,), jnp.float32)

        def zbody(i, carry):
            acc_v[pl.ds(i * _LANES, _LANES)] = zeros
            return carry

        lax.fori_loop(0, n_pad // _LANES, zbody, 0)

        def body(i, carry):
            idx = col_v[pl.ds(i * _LANES, _LANES)]
            w = ew_v[pl.ds(i * _LANES, _LANES)]
            plsc.addupdate_scatter(acc_v, [idx], w)
            return carry

        lax.fori_loop(0, epw // _LANES, body, 0)
        pltpu.sync_copy(acc_v, out_hbm.at[wid])

    return deg_kernel(col, ew)


def _edge_aggregate(g_t, row, col, ew, n_pad, feats):
    e = row.shape[0]
    nhalf = 2
    fpw = feats * nhalf // _NW
    ehalf_sz = e // nhalf
    eb = 3200
    nblk = ehalf_sz // eb
    npair = nblk // 2
    unroll = 2

    @functools.partial(
        pl.kernel,
        out_type=jax.ShapeDtypeStruct((nhalf, feats, n_pad), jnp.float32),
        mesh=_sc_mesh(),
        compiler_params=pltpu.CompilerParams(needs_layout_passes=False),
        scratch_types=(
            [pltpu.VMEM((n_pad,), jnp.float32) for _ in range(2 * fpw)]
            + [
                pltpu.VMEM((eb,), jnp.int32),
                pltpu.VMEM((eb,), jnp.int32),
                pltpu.VMEM((eb,), jnp.float32),
                pltpu.VMEM((eb,), jnp.int32),
                pltpu.VMEM((eb,), jnp.int32),
                pltpu.VMEM((eb,), jnp.float32),
                pltpu.SemaphoreType.DMA,
                pltpu.SemaphoreType.DMA,
            ]
        ),
    )
    def msg_kernel(g_hbm, row_hbm, col_hbm, ew_hbm, out_hbm, *scratch):
        g_vs = scratch[:fpw]
        acc_vs = scratch[fpw:2 * fpw]
        b0 = scratch[2 * fpw:2 * fpw + 3] + (scratch[2 * fpw + 6],)
        b1 = scratch[2 * fpw + 3:2 * fpw + 6] + (scratch[2 * fpw + 7],)
        bufs = (b0, b1)
        wid = lax.axis_index("s") * _NC + lax.axis_index("c")
        ehalf = wid % nhalf
        fbase = (wid // nhalf) * fpw
        ebase0 = ehalf * ehalf_sz
        for f in range(fpw):
            pltpu.sync_copy(g_hbm.at[fbase + f], g_vs[f])

        zeros = jnp.zeros((_LANES,), jnp.float32)

        def zbody(i, carry):
            for f in range(fpw):
                acc_vs[f][pl.ds(i * _LANES, _LANES)] = zeros
            return carry

        lax.fori_loop(0, n_pad // _LANES, zbody, 0)

        def start(slot, b):
            rv, cv, wv, sem = bufs[slot]
            ebase = ebase0 + b * eb
            pltpu.async_copy(row_hbm.at[pl.ds(ebase, eb)], rv, sem)
            pltpu.async_copy(col_hbm.at[pl.ds(ebase, eb)], cv, sem)
            pltpu.async_copy(ew_hbm.at[pl.ds(ebase, eb)], wv, sem)

        def wait(slot):
            rv, cv, wv, sem = bufs[slot]
            pltpu.make_async_copy(row_hbm.at[pl.ds(0, eb)], rv, sem).wait()
            pltpu.make_async_copy(col_hbm.at[pl.ds(0, eb)], cv, sem).wait()
            pltpu.make_async_copy(ew_hbm.at[pl.ds(0, eb)], wv, sem).wait()

        def process(slot):
            rv, cv, wv, _ = bufs[slot]

            @plsc.parallel_loop(0, eb // _LANES, 1, unroll=unroll)
            def _chunks(i):
                off = i * _LANES
                r = rv[pl.ds(off, _LANES)]
                c = cv[pl.ds(off, _LANES)]
                w = wv[pl.ds(off, _LANES)]
                for f in range(fpw):
                    v = plsc.load_gather(g_vs[f], [r])
                    plsc.addupdate_scatter(acc_vs[f], [c], v * w)

        start(0, 0)

        def pair(i, carry):
            start(1, 2 * i + 1)
            wait(0)
            process(0)

            @pl.when(i < npair - 1)
            def _():
                start(0, 2 * i + 2)

            wait(1)
            process(1)
            return carry

        lax.fori_loop(0, npair, pair, 0)
        for f in range(fpw):
            pltpu.sync_copy(acc_vs[f], out_hbm.at[ehalf, fbase + f])

    return msg_kernel(g_t, row, col, ew)


def _embed_matmul(x_t, w0_t, b0c, w1_t, n_pad):
    bn = 1280
    grid = n_pad // bn
    emb = w0_t.shape[0]
    feats = w1_t.shape[0]
    in_ch = x_t.shape[0]

    def body(x_ref, w0_ref, b0_ref, w1_ref, out_ref):
        h = jnp.dot(w0_ref[...], x_ref[...], preferred_element_type=jnp.float32)
        h = h + b0_ref[...]
        h = _SELU_SCALE * jnp.where(h > 0, h, _SELU_ALPHA * (jnp.exp(h) - 1.0))
        out_ref[...] = jnp.dot(w1_ref[...], h, preferred_element_type=jnp.float32)

    return pl.pallas_call(
        body,
        grid=(grid,),
        in_specs=[
            pl.BlockSpec((in_ch, bn), lambda i: (0, i)),
            pl.BlockSpec((emb, in_ch), lambda i: (0, 0)),
            pl.BlockSpec((emb, 1), lambda i: (0, 0)),
            pl.BlockSpec((feats, emb), lambda i: (0, 0)),
        ],
        out_specs=pl.BlockSpec((feats, bn), lambda i: (0, i)),
        out_shape=jax.ShapeDtypeStruct((feats, n_pad), jnp.float32),
    )(x_t, w0_t, b0c, w1_t)


def _dis_and_g1(parts, hw1_t):
    nw, n_pad = parts.shape
    feats = hw1_t.shape[0]

    def body(p_ref, h_ref, dis_ref, g_ref):
        deg = jnp.sum(p_ref[...], axis=0, keepdims=True) + 1.0
        dis = lax.rsqrt(deg)
        dis_ref[...] = dis
        g_ref[...] = h_ref[...] * dis

    return pl.pallas_call(
        body,
        out_shape=(
            jax.ShapeDtypeStruct((1, n_pad), jnp.float32),
            jax.ShapeDtypeStruct((feats, n_pad), jnp.float32),
        ),
    )(parts, hw1_t)


def _conv1_epilogue(s1_t, g1_t, dis, w2_t, b1c):
    feats, n_pad = g1_t.shape

    def body(s_ref, g_ref, d_ref, w2_ref, b1_ref, out_ref):
        d = d_ref[...]
        s = s_ref[0] + s_ref[1]
        out1 = d * (s + g_ref[...]) + b1_ref[...]
        hw2 = jnp.dot(w2_ref[...], out1, preferred_element_type=jnp.float32)
        out_ref[...] = hw2 * d

    return pl.pallas_call(
        body,
        out_shape=jax.ShapeDtypeStruct((feats, n_pad), jnp.float32),
    )(s1_t, g1_t, dis, w2_t, b1c)


def _conv2_softmax(s2_t, g2_t, dis, b2c):
    feats, n_pad = g2_t.shape

    def body(s_ref, g_ref, d_ref, b2_ref, out_ref):
        o = d_ref[...] * (s_ref[0] + s_ref[1] + g_ref[...]) + b2_ref[...]
        m = jnp.max(o, axis=0, keepdims=True)
        ex = jnp.exp(o - m)
        out_ref[...] = ex / jnp.sum(ex, axis=0, keepdims=True)

    return pl.pallas_call(
        body,
        out_shape=jax.ShapeDtypeStruct((feats, n_pad), jnp.float32),
    )(s2_t, g2_t, dis, b2c)


def kernel(x, edge_index, edge_attr, W0, b0, W1, b1, W2, b2):
    n = x.shape[0]
    n_pad = 10240
    feats = W1.shape[1]

    row = edge_index[0]
    col = edge_index[1]
    ew = edge_attr

    x_t = jnp.pad(x.T, ((0, 0), (0, n_pad - n)))
    w0_t = W0.T
    w1_t = W1.T
    w2_t = W2.T
    b0c = b0[:, None]
    b1c = b1[:, None]
    b2c = b2[:, None]

    parts = _deg_partials(col, ew, n_pad)
    hw1_t = _embed_matmul(x_t, w0_t, b0c, w1_t, n_pad)

    dis, g1_t = _dis_and_g1(parts, hw1_t)
    s1_t = _edge_aggregate(g1_t, row, col, ew, n_pad, feats)
    g2_t = _conv1_epilogue(s1_t, g1_t, dis, w2_t, b1c)
    s2_t = _edge_aggregate(g2_t, row, col, ew, n_pad, feats)
    out_t = _conv2_softmax(s2_t, g2_t, dis, b2c)

    return out_t[:, :n].T
